# baseline (speedup 1.0000x reference)
import functools

import jax
import jax.numpy as jnp
from jax import lax
from jax.experimental import pallas as pl
from jax.experimental.pallas import tpu as pltpu
from jax.experimental.pallas import tpu_sc as plsc

SD = 1024
TD = 2048
E = 8
K = 2
S = 2048
BLK = 256

_PREC = jax.lax.Precision.DEFAULT


def _dot(a, b):
    return jax.lax.dot_general(a, b, (((1,), (0,)), ((), ())),
                               precision=_PREC, preferred_element_type=jnp.float32)


def _ln(x, g, b):
    m = jnp.mean(x, axis=-1, keepdims=True)
    v = jnp.mean((x - m) ** 2, axis=-1, keepdims=True)
    return (x - m) / jnp.sqrt(v + 1e-5) * g + b


def _gelu(x):
    return 0.5 * x * (1.0 + jax.lax.erf(x * 0.7071067811865476))


def _bf(x):
    return x.astype(jnp.bfloat16)


def _scores_body(x_ref, t_ref, w1_ref, b1_ref, g1_ref, be1_ref, w2_ref, b2_ref,
                 g2_ref, be2_ref, w3_ref, b3_ref, tmp_ref, wp_ref, bp_ref,
                 wg1_ref, bg1_ref, gg_ref, beg_ref, wg2_ref, bg2_ref, ws1_ref,
                 bs1_ref, ws2_ref, bs2_ref, i0_ref, i1_ref, w0_ref, w1o_ref):
    x = x_ref[...]
    t = t_ref[...]
    h = _gelu(_ln(_dot(x, w1_ref[...]) + b1_ref[...], g1_ref[...], be1_ref[...]))
    h = _gelu(_ln(_dot(_bf(h), w2_ref[...]) + b2_ref[...], g2_ref[...], be2_ref[...]))
    cap = jax.nn.softmax((_dot(_bf(h), w3_ref[...]) + b3_ref[...]) / tmp_ref[...],
                         axis=-1)
    tproj = _dot(t, wp_ref[...]) + bp_ref[...]
    pre = (_dot(x, wg1_ref[0:SD, :]) + _dot(_bf(tproj), wg1_ref[SD:2 * SD, :])
           + bg1_ref[...])
    gh = _gelu(_ln(pre, gg_ref[...], beg_ref[...]))
    gap = jax.nn.softmax(_dot(_bf(gh), wg2_ref[...]) + bg2_ref[...], axis=-1)
    eih = _gelu(_dot(t, ws1_ref[...]) + bs1_ref[...])
    ei = jax.nn.softmax(_dot(_bf(eih), ws2_ref[...]) + bs2_ref[...], axis=-1)
    comb = 0.4 * cap + 0.3 * gap + 0.3 * ei
    lane = jax.lax.broadcasted_iota(jnp.int32, (BLK, E), 1)
    m1 = jnp.max(comb, axis=-1, keepdims=True)
    i1 = jnp.min(jnp.where(comb == m1, lane, E), axis=-1, keepdims=True)
    masked = jnp.where(lane == i1, -jnp.inf, comb)
    m2 = jnp.max(masked, axis=-1, keepdims=True)
    i2 = jnp.min(jnp.where(masked == m2, lane, E), axis=-1, keepdims=True)
    denom = m1 + m2 + 1e-8
    tok = pl.program_id(0) * BLK + jax.lax.broadcasted_iota(jnp.int32, (BLK, 1), 0)
    i0_ref[...] = i1 * S + tok
    i1_ref[...] = i2 * S + tok
    w0_ref[...] = m1 / denom
    w1o_ref[...] = m2 / denom


def _scores(x, t, W1, b1, g1, be1, W2, b2, g2, be2, W3, b3, temperature,
            Wp, bp, Wg1, bg1, gg, beg, Wg2, bg2, Ws1, bs1, Ws2, bs2):
    full = lambda shape: pl.BlockSpec(shape, lambda i: (0, 0))
    blk_col = lambda n: pl.BlockSpec((BLK, n), lambda i: (i, 0))
    r = lambda v: v.reshape(1, -1)
    return pl.pallas_call(
        _scores_body,
        grid=(S // BLK,),
        in_specs=[
            pl.BlockSpec((BLK, SD), lambda i: (i, 0)),
            pl.BlockSpec((BLK, TD), lambda i: (i, 0)),
            full((SD, 2 * SD)), full((1, 2 * SD)), full((1, 2 * SD)), full((1, 2 * SD)),
            full((2 * SD, SD)), full((1, SD)), full((1, SD)), full((1, SD)),
            full((SD, E)), full((1, E)), full((1, 1)),
            full((TD, SD)), full((1, SD)),
            full((2 * SD, TD)), full((1, TD)), full((1, TD)), full((1, TD)),
            full((TD, E)), full((1, E)),
            full((TD, TD // 2)), full((1, TD // 2)),
            full((TD // 2, E)), full((1, E)),
        ],
        out_specs=[blk_col(1), blk_col(1), blk_col(1), blk_col(1)],
        out_shape=[
            jax.ShapeDtypeStruct((S, 1), jnp.int32),
            jax.ShapeDtypeStruct((S, 1), jnp.int32),
            jax.ShapeDtypeStruct((S, 1), jnp.float32),
            jax.ShapeDtypeStruct((S, 1), jnp.float32),
        ],
    )(_bf(x), _bf(t), _bf(W1), r(b1), r(g1), r(be1), _bf(W2), r(b2), r(g2),
      r(be2), _bf(W3), r(b3), temperature.reshape(1, 1), _bf(Wp), r(bp),
      _bf(Wg1), r(bg1), r(gg), r(beg), _bf(Wg2), r(bg2), _bf(Ws1), r(bs1),
      _bf(Ws2), r(bs2))


NC = 2
NS = 16
NW = NC * NS
TPW = S // NW
CH = 8
NCH = TPW // CH


def _sc_combine_body(table_hbm, idx_hbm, w_hbm, out_hbm, idx_v, w_v, rows_v,
                     out_v, gsem0, gsem1, osem0, osem1):
    wid = lax.axis_index("s") * NC + lax.axis_index("c")
    ibase = wid * (2 * TPW)
    gsems = (gsem0, gsem1)
    osems = (osem0, osem1)
    pltpu.sync_copy(idx_hbm.at[pl.ds(ibase, 2 * TPW)], idx_v)
    pltpu.sync_copy(w_hbm.at[pl.ds(ibase, 2 * TPW)], w_v)
    gcp = [None, None]
    ocp = [None, None]
    iv0 = idx_v[pl.ds(0, 16)]
    gcp[0] = pltpu.async_copy(table_hbm.at[iv0], rows_v.at[0], gsems[0])
    for c in range(NCH):
        buf = c % 2
        nxt = (c + 1) % 2
        if c + 1 < NCH:
            ivn = idx_v[pl.ds((c + 1) * 16, 16)]
            gcp[nxt] = pltpu.async_copy(table_hbm.at[ivn], rows_v.at[nxt],
                                        gsems[nxt])
        gcp[buf].wait()
        if ocp[buf] is not None:
            ocp[buf].wait()
        for t in range(CH):
            w0 = w_v[c * 16 + 2 * t, :]
            w1 = w_v[c * 16 + 2 * t + 1, :]

            def body(j, _, buf=buf, t=t, w0=w0, w1=w1):
                a = rows_v[buf, 2 * t, pl.ds(j * 16, 16)]
                b = rows_v[buf, 2 * t + 1, pl.ds(j * 16, 16)]
                out_v[buf, t, pl.ds(j * 16, 16)] = a * w0 + b * w1
                return 0

            lax.fori_loop(0, TD // 16, body, 0)
        ocp[buf] = pltpu.async_copy(
            out_v.at[buf], out_hbm.at[pl.ds(wid * TPW + c * CH, CH)], osems[buf])
    for b in range(2):
        if ocp[b] is not None:
            ocp[b].wait()


def _combine_sc(table, idx_flat, w_flat):
    return pl.kernel(
        _sc_combine_body,
        out_type=jax.ShapeDtypeStruct((S, TD), jnp.float32),
        mesh=plsc.VectorSubcoreMesh(core_axis_name="c", subcore_axis_name="s"),
        scratch_types=[
            pltpu.VMEM((2 * TPW,), jnp.int32),
            pltpu.VMEM((2 * TPW, 16), jnp.float32),
            pltpu.VMEM((2, 2 * CH, TD), jnp.float32),
            pltpu.VMEM((2, CH, TD), jnp.float32),
            pltpu.SemaphoreType.DMA,
            pltpu.SemaphoreType.DMA,
            pltpu.SemaphoreType.DMA,
            pltpu.SemaphoreType.DMA,
        ],
    )(table, idx_flat, w_flat)


def kernel(student_hidden, teacher_expert_outputs, W1, b1, g1, be1, W2, b2, g2,
           be2, W3, b3, Wp, bp, Wg1, bg1, gg, beg, Wg2, bg2, temperature,
           Ws1, bs1, Ws2, bs2, Wr, br):
    x = student_hidden.reshape(S, SD)
    teacher = teacher_expert_outputs[0].reshape(S, TD)
    i0, i1, w0, w1 = _scores(x, teacher, W1, b1, g1, be1, W2, b2, g2, be2, W3,
                             b3, temperature, Wp, bp, Wg1, bg1, gg, beg, Wg2,
                             bg2, Ws1, bs1, Ws2, bs2)
    table = teacher_expert_outputs.reshape(E * S, TD)
    idx_flat = jnp.concatenate([i0, i1], axis=1).reshape(2 * S)
    w_flat = jnp.concatenate([w0, w1], axis=1).reshape(2 * S)
    w_bcast = jnp.broadcast_to(w_flat[:, None], (2 * S, 16))
    routed = _combine_sc(table, idx_flat, w_bcast)
    return routed.reshape(1, S, TD)


# speedup vs baseline: 1.0754x; 1.0754x over previous
import functools

import jax
import jax.numpy as jnp
from jax import lax
from jax.experimental import pallas as pl
from jax.experimental.pallas import tpu as pltpu
from jax.experimental.pallas import tpu_sc as plsc

SD = 1024
TD = 2048
E = 8
K = 2
S = 2048
BLK = 256

_PREC = jax.lax.Precision.DEFAULT


def _dot(a, b):
    return jax.lax.dot_general(a, b, (((1,), (0,)), ((), ())),
                               precision=_PREC, preferred_element_type=jnp.float32)


def _ln(x, g, b):
    m = jnp.mean(x, axis=-1, keepdims=True)
    v = jnp.mean((x - m) ** 2, axis=-1, keepdims=True)
    return (x - m) / jnp.sqrt(v + 1e-5) * g + b


def _gelu(x):
    return 0.5 * x * (1.0 + jax.lax.erf(x * 0.7071067811865476))


def _bf(x):
    return x.astype(jnp.bfloat16)


def _scores_body(x_ref, t_ref, w1_ref, b1_ref, g1_ref, be1_ref, w2_ref, b2_ref,
                 g2_ref, be2_ref, w3_ref, b3_ref, tmp_ref, wp_ref, bp_ref,
                 wg1_ref, bg1_ref, gg_ref, beg_ref, wg2_ref, bg2_ref, ws1_ref,
                 bs1_ref, ws2_ref, bs2_ref, i0_ref, i1_ref, w0_ref, w1o_ref):
    x = x_ref[...]
    t = t_ref[...]
    h = _gelu(_ln(_dot(x, w1_ref[...]) + b1_ref[...], g1_ref[...], be1_ref[...]))
    h = _gelu(_ln(_dot(_bf(h), w2_ref[...]) + b2_ref[...], g2_ref[...], be2_ref[...]))
    cap = jax.nn.softmax((_dot(_bf(h), w3_ref[...]) + b3_ref[...]) / tmp_ref[...],
                         axis=-1)
    tproj = _dot(t, wp_ref[...]) + bp_ref[...]
    pre = (_dot(x, wg1_ref[0:SD, :]) + _dot(_bf(tproj), wg1_ref[SD:2 * SD, :])
           + bg1_ref[...])
    gh = _gelu(_ln(pre, gg_ref[...], beg_ref[...]))
    gap = jax.nn.softmax(_dot(_bf(gh), wg2_ref[...]) + bg2_ref[...], axis=-1)
    eih = _gelu(_dot(t, ws1_ref[...]) + bs1_ref[...])
    ei = jax.nn.softmax(_dot(_bf(eih), ws2_ref[...]) + bs2_ref[...], axis=-1)
    comb = 0.4 * cap + 0.3 * gap + 0.3 * ei
    lane = jax.lax.broadcasted_iota(jnp.int32, (BLK, E), 1)
    m1 = jnp.max(comb, axis=-1, keepdims=True)
    i1 = jnp.min(jnp.where(comb == m1, lane, E), axis=-1, keepdims=True)
    masked = jnp.where(lane == i1, -jnp.inf, comb)
    m2 = jnp.max(masked, axis=-1, keepdims=True)
    i2 = jnp.min(jnp.where(masked == m2, lane, E), axis=-1, keepdims=True)
    denom = m1 + m2 + 1e-8
    tok = pl.program_id(0) * BLK + jax.lax.broadcasted_iota(jnp.int32, (BLK, 1), 0)
    i0_ref[...] = i1 * S + tok
    i1_ref[...] = i2 * S + tok
    w0_ref[...] = m1 / denom
    w1o_ref[...] = m2 / denom


def _scores(x, t, W1, b1, g1, be1, W2, b2, g2, be2, W3, b3, temperature,
            Wp, bp, Wg1, bg1, gg, beg, Wg2, bg2, Ws1, bs1, Ws2, bs2):
    full = lambda shape: pl.BlockSpec(shape, lambda i: (0, 0))
    blk_col = lambda n: pl.BlockSpec((BLK, n), lambda i: (i, 0))
    r = lambda v: v.reshape(1, -1)
    return pl.pallas_call(
        _scores_body,
        grid=(S // BLK,),
        in_specs=[
            pl.BlockSpec((BLK, SD), lambda i: (i, 0)),
            pl.BlockSpec((BLK, TD), lambda i: (i, 0)),
            full((SD, 2 * SD)), full((1, 2 * SD)), full((1, 2 * SD)), full((1, 2 * SD)),
            full((2 * SD, SD)), full((1, SD)), full((1, SD)), full((1, SD)),
            full((SD, E)), full((1, E)), full((1, 1)),
            full((TD, SD)), full((1, SD)),
            full((2 * SD, TD)), full((1, TD)), full((1, TD)), full((1, TD)),
            full((TD, E)), full((1, E)),
            pl.BlockSpec((TD, TD // 2), lambda i: (0, 0)), full((1, TD // 2)),
            full((TD // 2, E)), full((1, E)),
        ],
        out_specs=[blk_col(1), blk_col(1), blk_col(1), blk_col(1)],
        out_shape=[
            jax.ShapeDtypeStruct((S, 1), jnp.int32),
            jax.ShapeDtypeStruct((S, 1), jnp.int32),
            jax.ShapeDtypeStruct((S, 1), jnp.float32),
            jax.ShapeDtypeStruct((S, 1), jnp.float32),
        ],
    )(x, t, W1, r(b1), r(g1), r(be1), W2, r(b2), r(g2),
      r(be2), W3, r(b3), temperature.reshape(1, 1), Wp, r(bp),
      Wg1, r(bg1), r(gg), r(beg), Wg2, r(bg2), _bf(Ws1), r(bs1),
      Ws2, r(bs2))


NC = 2
NS = 16
NW = NC * NS
TPW = S // NW
CH = 8
NCH = TPW // CH


def _sc_combine_body(table_hbm, idx_hbm, w_hbm, out_hbm, idx_v, w_v, rows_v,
                     out_v, gsem0, gsem1, osem0, osem1):
    wid = lax.axis_index("s") * NC + lax.axis_index("c")
    ibase = wid * (2 * TPW)
    gsems = (gsem0, gsem1)
    osems = (osem0, osem1)
    pltpu.sync_copy(idx_hbm.at[pl.ds(ibase, 2 * TPW)], idx_v)
    pltpu.sync_copy(w_hbm.at[pl.ds(ibase, 2 * TPW)], w_v)
    gcp = [None, None]
    ocp = [None, None]
    iv0 = idx_v[pl.ds(0, 16)]
    gcp[0] = pltpu.async_copy(table_hbm.at[iv0], rows_v.at[0], gsems[0])
    for c in range(NCH):
        buf = c % 2
        nxt = (c + 1) % 2
        if c + 1 < NCH:
            ivn = idx_v[pl.ds((c + 1) * 16, 16)]
            gcp[nxt] = pltpu.async_copy(table_hbm.at[ivn], rows_v.at[nxt],
                                        gsems[nxt])
        gcp[buf].wait()
        if ocp[buf] is not None:
            ocp[buf].wait()
        for t in range(CH):
            w0 = w_v[c * 16 + 2 * t, :]
            w1 = w_v[c * 16 + 2 * t + 1, :]

            def body(j, _, buf=buf, t=t, w0=w0, w1=w1):
                a = rows_v[buf, 2 * t, pl.ds(j * 16, 16)]
                b = rows_v[buf, 2 * t + 1, pl.ds(j * 16, 16)]
                out_v[buf, t, pl.ds(j * 16, 16)] = a * w0 + b * w1
                return 0

            lax.fori_loop(0, TD // 16, body, 0)
        ocp[buf] = pltpu.async_copy(
            out_v.at[buf], out_hbm.at[pl.ds(wid * TPW + c * CH, CH)], osems[buf])
    for b in range(2):
        if ocp[b] is not None:
            ocp[b].wait()


def _combine_sc(table, idx_flat, w_flat):
    return pl.kernel(
        _sc_combine_body,
        out_type=jax.ShapeDtypeStruct((S, TD), jnp.float32),
        mesh=plsc.VectorSubcoreMesh(core_axis_name="c", subcore_axis_name="s"),
        scratch_types=[
            pltpu.VMEM((2 * TPW,), jnp.int32),
            pltpu.VMEM((2 * TPW, 16), jnp.float32),
            pltpu.VMEM((2, 2 * CH, TD), jnp.float32),
            pltpu.VMEM((2, CH, TD), jnp.float32),
            pltpu.SemaphoreType.DMA,
            pltpu.SemaphoreType.DMA,
            pltpu.SemaphoreType.DMA,
            pltpu.SemaphoreType.DMA,
        ],
    )(table, idx_flat, w_flat)


def kernel(student_hidden, teacher_expert_outputs, W1, b1, g1, be1, W2, b2, g2,
           be2, W3, b3, Wp, bp, Wg1, bg1, gg, beg, Wg2, bg2, temperature,
           Ws1, bs1, Ws2, bs2, Wr, br):
    x = student_hidden.reshape(S, SD)
    teacher = teacher_expert_outputs[0].reshape(S, TD)
    i0, i1, w0, w1 = _scores(x, teacher, W1, b1, g1, be1, W2, b2, g2, be2, W3,
                             b3, temperature, Wp, bp, Wg1, bg1, gg, beg, Wg2,
                             bg2, Ws1, bs1, Ws2, bs2)
    table = teacher_expert_outputs.reshape(E * S, TD)
    idx_flat = jnp.concatenate([i0, i1], axis=1).reshape(2 * S)
    w_flat = jnp.concatenate([w0, w1], axis=1).reshape(2 * S)
    w_bcast = jnp.broadcast_to(w_flat[:, None], (2 * S, 16))
    routed = _combine_sc(table, idx_flat, w_bcast)
    return routed.reshape(1, S, TD)


# speedup vs baseline: 1.1628x; 1.0813x over previous
import functools

import jax
import jax.numpy as jnp
from jax import lax
from jax.experimental import pallas as pl
from jax.experimental.pallas import tpu as pltpu
from jax.experimental.pallas import tpu_sc as plsc

SD = 1024
TD = 2048
E = 8
K = 2
S = 2048
BLK = 256

_PREC = jax.lax.Precision.DEFAULT


def _dot(a, b):
    return jax.lax.dot_general(a, b, (((1,), (0,)), ((), ())),
                               precision=_PREC, preferred_element_type=jnp.float32)


def _ln(x, g, b):
    m = jnp.mean(x, axis=-1, keepdims=True)
    v = jnp.mean((x - m) ** 2, axis=-1, keepdims=True)
    return (x - m) / jnp.sqrt(v + 1e-5) * g + b


def _gelu(x):
    return 0.5 * x * (1.0 + jax.lax.erf(x * 0.7071067811865476))


def _bf(x):
    return x.astype(jnp.bfloat16)


def _scores_body(x_ref, t_ref, w1_ref, b1_ref, g1_ref, be1_ref, w2_ref, b2_ref,
                 g2_ref, be2_ref, w3_ref, b3_ref, tmp_ref, wp_ref, bp_ref,
                 wg1_ref, bg1_ref, gg_ref, beg_ref, wg2_ref, bg2_ref, ws1_ref,
                 bs1_ref, ws2_ref, bs2_ref, i0_ref, i1_ref, w0_ref, w1o_ref):
    x = x_ref[...]
    t = t_ref[...]
    h = _gelu(_ln(_dot(x, w1_ref[...]) + b1_ref[...], g1_ref[...], be1_ref[...]))
    h = _gelu(_ln(_dot(_bf(h), w2_ref[...]) + b2_ref[...], g2_ref[...], be2_ref[...]))
    cap = jax.nn.softmax((_dot(_bf(h), w3_ref[...]) + b3_ref[...]) / tmp_ref[...],
                         axis=-1)
    tproj = _dot(t, wp_ref[...]) + bp_ref[...]
    pre = (_dot(x, wg1_ref[0:SD, :]) + _dot(_bf(tproj), wg1_ref[SD:2 * SD, :])
           + bg1_ref[...])
    gh = _gelu(_ln(pre, gg_ref[...], beg_ref[...]))
    gap = jax.nn.softmax(_dot(_bf(gh), wg2_ref[...]) + bg2_ref[...], axis=-1)
    eih = _gelu(_dot(t, ws1_ref[...]) + bs1_ref[...])
    ei = jax.nn.softmax(_dot(_bf(eih), ws2_ref[...]) + bs2_ref[...], axis=-1)
    comb = 0.4 * cap + 0.3 * gap + 0.3 * ei
    lane = jax.lax.broadcasted_iota(jnp.int32, (BLK, E), 1)
    m1 = jnp.max(comb, axis=-1, keepdims=True)
    i1 = jnp.min(jnp.where(comb == m1, lane, E), axis=-1, keepdims=True)
    masked = jnp.where(lane == i1, -jnp.inf, comb)
    m2 = jnp.max(masked, axis=-1, keepdims=True)
    i2 = jnp.min(jnp.where(masked == m2, lane, E), axis=-1, keepdims=True)
    denom = m1 + m2 + 1e-8
    tok = pl.program_id(0) * BLK + jax.lax.broadcasted_iota(jnp.int32, (BLK, 1), 0)
    i0_ref[...] = i1 * S + tok
    i1_ref[...] = i2 * S + tok
    w0_ref[...] = m1 / denom
    w1o_ref[...] = m2 / denom


def _scores(x, t, W1, b1, g1, be1, W2, b2, g2, be2, W3, b3, temperature,
            Wp, bp, Wg1, bg1, gg, beg, Wg2, bg2, Ws1, bs1, Ws2, bs2):
    full = lambda shape: pl.BlockSpec(shape, lambda i: (0, 0))
    blk_col = lambda n: pl.BlockSpec((BLK, n), lambda i: (i, 0))
    r = lambda v: v.reshape(1, -1)
    return pl.pallas_call(
        _scores_body,
        grid=(S // BLK,),
        in_specs=[
            pl.BlockSpec((BLK, SD), lambda i: (i, 0)),
            pl.BlockSpec((BLK, TD), lambda i: (i, 0)),
            full((SD, 2 * SD)), full((1, 2 * SD)), full((1, 2 * SD)), full((1, 2 * SD)),
            full((2 * SD, SD)), full((1, SD)), full((1, SD)), full((1, SD)),
            full((SD, E)), full((1, E)), full((1, 1)),
            full((TD, SD)), full((1, SD)),
            full((2 * SD, TD)), full((1, TD)), full((1, TD)), full((1, TD)),
            full((TD, E)), full((1, E)),
            pl.BlockSpec((TD, TD // 2), lambda i: (0, 0)), full((1, TD // 2)),
            full((TD // 2, E)), full((1, E)),
        ],
        out_specs=[blk_col(1), blk_col(1), blk_col(1), blk_col(1)],
        out_shape=[
            jax.ShapeDtypeStruct((S, 1), jnp.int32),
            jax.ShapeDtypeStruct((S, 1), jnp.int32),
            jax.ShapeDtypeStruct((S, 1), jnp.float32),
            jax.ShapeDtypeStruct((S, 1), jnp.float32),
        ],
    )(x, t, W1, r(b1), r(g1), r(be1), W2, r(b2), r(g2),
      r(be2), W3, r(b3), temperature.reshape(1, 1), Wp, r(bp),
      Wg1, r(bg1), r(gg), r(beg), Wg2, r(bg2), _bf(Ws1), r(bs1),
      Ws2, r(bs2))


NC = 2
NS = 16
NW = NC * NS
TPW = S // NW
CH = 8
NCH = TPW // CH


def _sc_combine_body(table_hbm, idx_hbm, w_hbm, out_hbm, idx_v, w_v, rows_v,
                     out_v, gsem0, gsem1, osem0, osem1):
    wid = lax.axis_index("s") * NC + lax.axis_index("c")
    ibase = wid * (2 * TPW)
    gsems = (gsem0, gsem1)
    osems = (osem0, osem1)
    pltpu.sync_copy(idx_hbm.at[pl.ds(ibase, 2 * TPW)], idx_v)
    pltpu.sync_copy(w_hbm.at[pl.ds(ibase, 2 * TPW)], w_v)
    gcp = [None, None]
    ocp = [None, None]
    iv0 = idx_v[pl.ds(0, 16)]
    gcp[0] = pltpu.async_copy(table_hbm.at[iv0], rows_v.at[0], gsems[0])
    for c in range(NCH):
        buf = c % 2
        nxt = (c + 1) % 2
        if c + 1 < NCH:
            ivn = idx_v[pl.ds((c + 1) * 16, 16)]
            gcp[nxt] = pltpu.async_copy(table_hbm.at[ivn], rows_v.at[nxt],
                                        gsems[nxt])
        gcp[buf].wait()
        if ocp[buf] is not None:
            ocp[buf].wait()
        ws = [w_v[c * 16 + k, :] for k in range(2 * CH)]

        def body(j, _, buf=buf, ws=ws):
            base = j * 16
            for t in range(CH):
                a = rows_v[buf, 2 * t, pl.ds(base, 16)]
                b = rows_v[buf, 2 * t + 1, pl.ds(base, 16)]
                out_v[buf, t, pl.ds(base, 16)] = a * ws[2 * t] + b * ws[2 * t + 1]
            return 0

        lax.fori_loop(0, TD // 16, body, 0)
        ocp[buf] = pltpu.async_copy(
            out_v.at[buf], out_hbm.at[pl.ds(wid * TPW + c * CH, CH)], osems[buf])
    for b in range(2):
        if ocp[b] is not None:
            ocp[b].wait()


def _combine_sc(table, idx_flat, w_flat):
    return pl.kernel(
        _sc_combine_body,
        out_type=jax.ShapeDtypeStruct((S, TD), jnp.float32),
        mesh=plsc.VectorSubcoreMesh(core_axis_name="c", subcore_axis_name="s"),
        scratch_types=[
            pltpu.VMEM((2 * TPW,), jnp.int32),
            pltpu.VMEM((2 * TPW, 16), jnp.float32),
            pltpu.VMEM((2, 2 * CH, TD), jnp.float32),
            pltpu.VMEM((2, CH, TD), jnp.float32),
            pltpu.SemaphoreType.DMA,
            pltpu.SemaphoreType.DMA,
            pltpu.SemaphoreType.DMA,
            pltpu.SemaphoreType.DMA,
        ],
    )(table, idx_flat, w_flat)


def kernel(student_hidden, teacher_expert_outputs, W1, b1, g1, be1, W2, b2, g2,
           be2, W3, b3, Wp, bp, Wg1, bg1, gg, beg, Wg2, bg2, temperature,
           Ws1, bs1, Ws2, bs2, Wr, br):
    x = student_hidden.reshape(S, SD)
    teacher = teacher_expert_outputs[0].reshape(S, TD)
    i0, i1, w0, w1 = _scores(x, teacher, W1, b1, g1, be1, W2, b2, g2, be2, W3,
                             b3, temperature, Wp, bp, Wg1, bg1, gg, beg, Wg2,
                             bg2, Ws1, bs1, Ws2, bs2)
    table = teacher_expert_outputs.reshape(E * S, TD)
    idx_flat = jnp.concatenate([i0, i1], axis=1).reshape(2 * S)
    w_flat = jnp.concatenate([w0, w1], axis=1).reshape(2 * S)
    w_bcast = jnp.broadcast_to(w_flat[:, None], (2 * S, 16))
    routed = _combine_sc(table, idx_flat, w_bcast)
    return routed.reshape(1, S, TD)


# speedup vs baseline: 1.1914x; 1.0246x over previous
import functools

import jax
import jax.numpy as jnp
from jax import lax
from jax.experimental import pallas as pl
from jax.experimental.pallas import tpu as pltpu
from jax.experimental.pallas import tpu_sc as plsc

SD = 1024
TD = 2048
E = 8
K = 2
S = 2048
BLK = 256

_PREC = jax.lax.Precision.DEFAULT


def _dot(a, b):
    return jax.lax.dot_general(a, b, (((1,), (0,)), ((), ())),
                               precision=_PREC, preferred_element_type=jnp.float32)


def _ln(x, g, b):
    m = jnp.mean(x, axis=-1, keepdims=True)
    v = jnp.mean((x - m) ** 2, axis=-1, keepdims=True)
    return (x - m) / jnp.sqrt(v + 1e-5) * g + b


def _gelu(x):
    return 0.5 * x * (1.0 + jax.lax.erf(x * 0.7071067811865476))


def _bf(x):
    return x.astype(jnp.bfloat16)


def _scores_body(x_ref, t_ref, w1_ref, b1_ref, g1_ref, be1_ref, w2_ref, b2_ref,
                 g2_ref, be2_ref, w3_ref, b3_ref, tmp_ref, wp_ref, bp_ref,
                 wg1_ref, bg1_ref, gg_ref, beg_ref, wg2_ref, bg2_ref, ws1_ref,
                 bs1_ref, ws2_ref, bs2_ref, i0_ref, i1_ref, w0_ref, w1o_ref):
    x = x_ref[...]
    t = t_ref[...]
    h = _gelu(_ln(_dot(x, w1_ref[...]) + b1_ref[...], g1_ref[...], be1_ref[...]))
    h = _gelu(_ln(_dot(_bf(h), w2_ref[...]) + b2_ref[...], g2_ref[...], be2_ref[...]))
    cap = jax.nn.softmax((_dot(_bf(h), w3_ref[...]) + b3_ref[...]) / tmp_ref[...],
                         axis=-1)
    tproj = _dot(t, wp_ref[...]) + bp_ref[...]
    pre = (_dot(x, wg1_ref[0:SD, :]) + _dot(_bf(tproj), wg1_ref[SD:2 * SD, :])
           + bg1_ref[...])
    gh = _gelu(_ln(pre, gg_ref[...], beg_ref[...]))
    gap = jax.nn.softmax(_dot(_bf(gh), wg2_ref[...]) + bg2_ref[...], axis=-1)
    eih = _gelu(_dot(t, ws1_ref[...]) + bs1_ref[...])
    ei = jax.nn.softmax(_dot(_bf(eih), ws2_ref[...]) + bs2_ref[...], axis=-1)
    comb = 0.4 * cap + 0.3 * gap + 0.3 * ei
    lane = jax.lax.broadcasted_iota(jnp.int32, (BLK, E), 1)
    m1 = jnp.max(comb, axis=-1, keepdims=True)
    i1 = jnp.min(jnp.where(comb == m1, lane, E), axis=-1, keepdims=True)
    masked = jnp.where(lane == i1, -jnp.inf, comb)
    m2 = jnp.max(masked, axis=-1, keepdims=True)
    i2 = jnp.min(jnp.where(masked == m2, lane, E), axis=-1, keepdims=True)
    denom = m1 + m2 + 1e-8
    tok = pl.program_id(0) * BLK + jax.lax.broadcasted_iota(jnp.int32, (BLK, 1), 0)
    i0_ref[...] = i1 * S + tok
    i1_ref[...] = i2 * S + tok
    w0_ref[...] = m1 / denom
    w1o_ref[...] = m2 / denom


def _scores(x, t, W1, b1, g1, be1, W2, b2, g2, be2, W3, b3, temperature,
            Wp, bp, Wg1, bg1, gg, beg, Wg2, bg2, Ws1, bs1, Ws2, bs2):
    full = lambda shape: pl.BlockSpec(shape, lambda i: (0, 0))
    blk_col = lambda n: pl.BlockSpec((BLK, n), lambda i: (i, 0))
    r = lambda v: v.reshape(1, -1)
    return pl.pallas_call(
        _scores_body,
        grid=(S // BLK,),
        in_specs=[
            pl.BlockSpec((BLK, SD), lambda i: (i, 0)),
            pl.BlockSpec((BLK, TD), lambda i: (i, 0)),
            full((SD, 2 * SD)), full((1, 2 * SD)), full((1, 2 * SD)), full((1, 2 * SD)),
            full((2 * SD, SD)), full((1, SD)), full((1, SD)), full((1, SD)),
            full((SD, E)), full((1, E)), full((1, 1)),
            full((TD, SD)), full((1, SD)),
            full((2 * SD, TD)), full((1, TD)), full((1, TD)), full((1, TD)),
            full((TD, E)), full((1, E)),
            pl.BlockSpec((TD, TD // 2), lambda i: (0, 0)), full((1, TD // 2)),
            full((TD // 2, E)), full((1, E)),
        ],
        out_specs=[blk_col(1), blk_col(1), blk_col(1), blk_col(1)],
        out_shape=[
            jax.ShapeDtypeStruct((S, 1), jnp.int32),
            jax.ShapeDtypeStruct((S, 1), jnp.int32),
            jax.ShapeDtypeStruct((S, 1), jnp.float32),
            jax.ShapeDtypeStruct((S, 1), jnp.float32),
        ],
    )(x, t, W1, r(b1), r(g1), r(be1), W2, r(b2), r(g2),
      r(be2), W3, r(b3), temperature.reshape(1, 1), Wp, r(bp),
      Wg1, r(bg1), r(gg), r(beg), Wg2, r(bg2), _bf(Ws1), r(bs1),
      Ws2, r(bs2))


NC = 2
NS = 16
NW = NC * NS
TPW = S // NW
CH = 8
NCH = TPW // CH
NBUF = 3


def _sc_combine_body(table_hbm, idx_hbm, w_hbm, out_hbm, idx_v, w_v, rows_v,
                     gsem0, gsem1, gsem2, osem0, osem1, osem2):
    wid = lax.axis_index("s") * NC + lax.axis_index("c")
    ibase = wid * (2 * TPW)
    gsems = (gsem0, gsem1, gsem2)
    osems = (osem0, osem1, osem2)
    pltpu.sync_copy(idx_hbm.at[pl.ds(ibase, 2 * TPW)], idx_v)
    pltpu.sync_copy(w_hbm.at[pl.ds(ibase, 2 * TPW)], w_v)
    gcp = [None] * NBUF
    ocp = [None] * NBUF
    for p in range(NBUF - 1):
        ivp = idx_v[pl.ds(p * 16, 16)]
        gcp[p] = pltpu.async_copy(table_hbm.at[ivp], rows_v.at[p], gsems[p])
    for c in range(NCH):
        buf = c % NBUF
        if c + NBUF - 1 < NCH:
            nxt = (c + NBUF - 1) % NBUF
            if ocp[nxt] is not None:
                ocp[nxt].wait()
            ivn = idx_v[pl.ds((c + NBUF - 1) * 16, 16)]
            gcp[nxt] = pltpu.async_copy(table_hbm.at[ivn], rows_v.at[nxt],
                                        gsems[nxt])
        gcp[buf].wait()
        ws = [w_v[c * 16 + k, :] for k in range(2 * CH)]

        def body(j, _, buf=buf, ws=ws):
            base = j * 16
            for t in range(CH):
                a = rows_v[buf, 2 * t, pl.ds(base, 16)]
                b = rows_v[buf, 2 * t + 1, pl.ds(base, 16)]
                rows_v[buf, t, pl.ds(base, 16)] = (a * ws[2 * t]
                                                   + b * ws[2 * t + 1])
            return 0

        lax.fori_loop(0, TD // 16, body, 0)
        ocp[buf] = pltpu.async_copy(
            rows_v.at[buf].at[pl.ds(0, CH)],
            out_hbm.at[pl.ds(wid * TPW + c * CH, CH)], osems[buf])
    for b in range(NBUF):
        if ocp[b] is not None:
            ocp[b].wait()


def _combine_sc(table, idx_flat, w_flat):
    return pl.kernel(
        _sc_combine_body,
        out_type=jax.ShapeDtypeStruct((S, TD), jnp.float32),
        mesh=plsc.VectorSubcoreMesh(core_axis_name="c", subcore_axis_name="s"),
        scratch_types=[
            pltpu.VMEM((2 * TPW,), jnp.int32),
            pltpu.VMEM((2 * TPW, 16), jnp.float32),
            pltpu.VMEM((NBUF, 2 * CH, TD), jnp.float32),
            pltpu.SemaphoreType.DMA,
            pltpu.SemaphoreType.DMA,
            pltpu.SemaphoreType.DMA,
            pltpu.SemaphoreType.DMA,
            pltpu.SemaphoreType.DMA,
            pltpu.SemaphoreType.DMA,
        ],
    )(table, idx_flat, w_flat)


def kernel(student_hidden, teacher_expert_outputs, W1, b1, g1, be1, W2, b2, g2,
           be2, W3, b3, Wp, bp, Wg1, bg1, gg, beg, Wg2, bg2, temperature,
           Ws1, bs1, Ws2, bs2, Wr, br):
    x = student_hidden.reshape(S, SD)
    teacher = teacher_expert_outputs[0].reshape(S, TD)
    i0, i1, w0, w1 = _scores(x, teacher, W1, b1, g1, be1, W2, b2, g2, be2, W3,
                             b3, temperature, Wp, bp, Wg1, bg1, gg, beg, Wg2,
                             bg2, Ws1, bs1, Ws2, bs2)
    table = teacher_expert_outputs.reshape(E * S, TD)
    idx_flat = jnp.concatenate([i0, i1], axis=1).reshape(2 * S)
    w_flat = jnp.concatenate([w0, w1], axis=1).reshape(2 * S)
    w_bcast = jnp.broadcast_to(w_flat[:, None], (2 * S, 16))
    routed = _combine_sc(table, idx_flat, w_bcast)
    return routed.reshape(1, S, TD)


# speedup vs baseline: 1.1934x; 1.0016x over previous
import jax
import jax.numpy as jnp
from jax import lax
from jax.experimental import pallas as pl
from jax.experimental.pallas import tpu as pltpu
from jax.experimental.pallas import tpu_sc as plsc

SD = 1024
TD = 2048
E = 8
K = 2
S = 2048
BLK = 256

_PREC = jax.lax.Precision.DEFAULT


def _dot(a, b):
    return jax.lax.dot_general(a, b, (((1,), (0,)), ((), ())),
                               precision=_PREC, preferred_element_type=jnp.float32)


def _ln(x, g, b):
    m = jnp.mean(x, axis=-1, keepdims=True)
    v = jnp.mean((x - m) ** 2, axis=-1, keepdims=True)
    return (x - m) / jnp.sqrt(v + 1e-5) * g + b


def _gelu(x):
    return 0.5 * x * (1.0 + jax.lax.erf(x * 0.7071067811865476))


def _bf(x):
    return x.astype(jnp.bfloat16)


def _scores_body(x_ref, t_ref, w1_ref, b1_ref, g1_ref, be1_ref, w2_ref, b2_ref,
                 g2_ref, be2_ref, w3_ref, b3_ref, tmp_ref, wp_ref, bp_ref,
                 wg1_ref, bg1_ref, gg_ref, beg_ref, wg2_ref, bg2_ref, ws1_ref,
                 bs1_ref, ws2_ref, bs2_ref, i0_ref, i1_ref, w0_ref, w1o_ref):
    x = x_ref[...]
    t = t_ref[...]
    h = _gelu(_ln(_dot(x, w1_ref[...]) + b1_ref[...], g1_ref[...], be1_ref[...]))
    h = _gelu(_ln(_dot(_bf(h), w2_ref[...]) + b2_ref[...], g2_ref[...], be2_ref[...]))
    cap = jax.nn.softmax((_dot(_bf(h), w3_ref[...]) + b3_ref[...]) / tmp_ref[...],
                         axis=-1)
    tproj = _dot(t, wp_ref[...]) + bp_ref[...]
    pre = (_dot(x, wg1_ref[0:SD, :]) + _dot(_bf(tproj), wg1_ref[SD:2 * SD, :])
           + bg1_ref[...])
    gh = _gelu(_ln(pre, gg_ref[...], beg_ref[...]))
    gap = jax.nn.softmax(_dot(_bf(gh), wg2_ref[...]) + bg2_ref[...], axis=-1)
    eih = _gelu(_dot(t, ws1_ref[...]) + bs1_ref[...])
    ei = jax.nn.softmax(_dot(_bf(eih), ws2_ref[...]) + bs2_ref[...], axis=-1)
    comb = 0.4 * cap + 0.3 * gap + 0.3 * ei
    lane = jax.lax.broadcasted_iota(jnp.int32, (BLK, E), 1)
    m1 = jnp.max(comb, axis=-1, keepdims=True)
    i1 = jnp.min(jnp.where(comb == m1, lane, E), axis=-1, keepdims=True)
    masked = jnp.where(lane == i1, -jnp.inf, comb)
    m2 = jnp.max(masked, axis=-1, keepdims=True)
    i2 = jnp.min(jnp.where(masked == m2, lane, E), axis=-1, keepdims=True)
    denom = m1 + m2 + 1e-8
    tok = pl.program_id(0) * BLK + jax.lax.broadcasted_iota(jnp.int32, (BLK, 1), 0)
    i0_ref[...] = i1 * S + tok
    i1_ref[...] = i2 * S + tok
    w0_ref[...] = m1 / denom
    w1o_ref[...] = m2 / denom


def _scores(x, t, W1, b1, g1, be1, W2, b2, g2, be2, W3, b3, temperature,
            Wp, bp, Wg1, bg1, gg, beg, Wg2, bg2, Ws1, bs1, Ws2, bs2):
    full = lambda shape: pl.BlockSpec(shape, lambda i: (0, 0))
    blk_col = lambda n: pl.BlockSpec((BLK, n), lambda i: (i, 0))
    r = lambda v: v.reshape(1, -1)
    return pl.pallas_call(
        _scores_body,
        grid=(S // BLK,),
        in_specs=[
            pl.BlockSpec((BLK, SD), lambda i: (i, 0)),
            pl.BlockSpec((BLK, TD), lambda i: (i, 0)),
            full((SD, 2 * SD)), full((1, 2 * SD)), full((1, 2 * SD)), full((1, 2 * SD)),
            full((2 * SD, SD)), full((1, SD)), full((1, SD)), full((1, SD)),
            full((SD, E)), full((1, E)), full((1, 1)),
            full((TD, SD)), full((1, SD)),
            full((2 * SD, TD)), full((1, TD)), full((1, TD)), full((1, TD)),
            full((TD, E)), full((1, E)),
            pl.BlockSpec((TD, TD // 2), lambda i: (0, 0)), full((1, TD // 2)),
            full((TD // 2, E)), full((1, E)),
        ],
        out_specs=[blk_col(1), blk_col(1), blk_col(1), blk_col(1)],
        out_shape=[
            jax.ShapeDtypeStruct((S, 1), jnp.int32),
            jax.ShapeDtypeStruct((S, 1), jnp.int32),
            jax.ShapeDtypeStruct((S, 1), jnp.float32),
            jax.ShapeDtypeStruct((S, 1), jnp.float32),
        ],
    )(x, t, W1, r(b1), r(g1), r(be1), W2, r(b2), r(g2),
      r(be2), W3, r(b3), temperature.reshape(1, 1), Wp, r(bp),
      Wg1, r(bg1), r(gg), r(beg), Wg2, r(bg2), _bf(Ws1), r(bs1),
      Ws2, r(bs2))


NC = 2
NS = 16
NW = NC * NS
TPW = S // NW
CH = 8
NCH = TPW // CH
NBUF = 3


def _sc_combine_body(table_hbm, idx_hbm, w_hbm, out_hbm, idx_v, w_v, rows_v,
                     gsem0, gsem1, gsem2, osem0, osem1, osem2):
    wid = lax.axis_index("s") * NC + lax.axis_index("c")
    ibase = wid * (2 * TPW)
    gsems = (gsem0, gsem1, gsem2)
    osems = (osem0, osem1, osem2)
    pltpu.sync_copy(idx_hbm.at[pl.ds(ibase, 2 * TPW)], idx_v)
    pltpu.sync_copy(w_hbm.at[pl.ds(ibase, 2 * TPW)], w_v)
    gcp = [None] * NBUF
    ocp = [None] * NBUF
    for p in range(NBUF - 1):
        ivp = idx_v[pl.ds(p * 16, 16)]
        gcp[p] = pltpu.async_copy(table_hbm.at[ivp], rows_v.at[p], gsems[p])
    for c in range(NCH):
        buf = c % NBUF
        if c + NBUF - 1 < NCH:
            nxt = (c + NBUF - 1) % NBUF
            if ocp[nxt] is not None:
                ocp[nxt].wait()
            ivn = idx_v[pl.ds((c + NBUF - 1) * 16, 16)]
            gcp[nxt] = pltpu.async_copy(table_hbm.at[ivn], rows_v.at[nxt],
                                        gsems[nxt])
        gcp[buf].wait()
        ws = [w_v[c * 16 + k, :] for k in range(2 * CH)]

        def body(j, _, buf=buf, ws=ws):
            base = j * 16
            for t in range(CH):
                a = rows_v[buf, 2 * t, pl.ds(base, 16)]
                b = rows_v[buf, 2 * t + 1, pl.ds(base, 16)]
                rows_v[buf, t, pl.ds(base, 16)] = (a * ws[2 * t]
                                                   + b * ws[2 * t + 1])
            return 0

        lax.fori_loop(0, TD // 16, body, 0)
        ocp[buf] = pltpu.async_copy(
            rows_v.at[buf].at[pl.ds(0, CH)],
            out_hbm.at[pl.ds(wid * TPW + c * CH, CH)], osems[buf])
    for b in range(NBUF):
        if ocp[b] is not None:
            ocp[b].wait()


def _combine_sc(table, idx_flat, w_flat):
    return pl.kernel(
        _sc_combine_body,
        out_type=jax.ShapeDtypeStruct((S, TD), jnp.float32),
        mesh=plsc.VectorSubcoreMesh(core_axis_name="c", subcore_axis_name="s"),
        scratch_types=[
            pltpu.VMEM((2 * TPW,), jnp.int32),
            pltpu.VMEM((2 * TPW, 16), jnp.float32),
            pltpu.VMEM((NBUF, 2 * CH, TD), jnp.float32),
            pltpu.SemaphoreType.DMA,
            pltpu.SemaphoreType.DMA,
            pltpu.SemaphoreType.DMA,
            pltpu.SemaphoreType.DMA,
            pltpu.SemaphoreType.DMA,
            pltpu.SemaphoreType.DMA,
        ],
    )(table, idx_flat, w_flat)


def kernel(student_hidden, teacher_expert_outputs, W1, b1, g1, be1, W2, b2, g2,
           be2, W3, b3, Wp, bp, Wg1, bg1, gg, beg, Wg2, bg2, temperature,
           Ws1, bs1, Ws2, bs2, Wr, br):
    x = student_hidden.reshape(S, SD)
    teacher = teacher_expert_outputs[0].reshape(S, TD)
    i0, i1, w0, w1 = _scores(x, teacher, W1, b1, g1, be1, W2, b2, g2, be2, W3,
                             b3, temperature, Wp, bp, Wg1, bg1, gg, beg, Wg2,
                             bg2, Ws1, bs1, Ws2, bs2)
    table = teacher_expert_outputs.reshape(E * S, TD)
    idx_flat = jnp.concatenate([i0, i1], axis=1).reshape(2 * S)
    w_flat = jnp.concatenate([w0, w1], axis=1).reshape(2 * S)
    w_bcast = jnp.broadcast_to(w_flat[:, None], (2 * S, 16))
    routed = _combine_sc(table, idx_flat, w_bcast)
    return routed.reshape(1, S, TD)
